# baseline (speedup 1.0000x reference)
import functools

import jax
import jax.numpy as jnp
from jax import lax
from jax.experimental import pallas as pl
from jax.experimental.pallas import tpu as pltpu
from jax.experimental.pallas import tpu_sc as plsc

NC = 2
NS = 16
L = 16
NW = NC * NS
B = 16384
E = 128
NUSERS = 100100
BPW = B // NW
CH = 64
NCHUNK = BPW // CH
NBUF = 3
IDX_ROWS = B // 128
NSL = E // L
IPW = BPW // 128


def _sc_body(uidx, midx, uemb, memb, ubias, mbias,
             parts_out, ubg_out, mbg_out,
             idx_u, idx_m, u0, m0, u1, m1, u2, m2, bu_all, bm_all, acc_v,
             sem_u0, sem_m0, sem_u1, sem_m1, sem_u2, sem_m2, sem_b):
    wid = lax.axis_index("s") * NC + lax.axis_index("c")
    rbase = wid * IPW
    pltpu.sync_copy(uidx.at[pl.ds(rbase, IPW)], idx_u)
    pltpu.sync_copy(midx.at[pl.ds(rbase, IPW)], idx_m)

    ubufs = (u0, u1, u2)
    mbufs = (m0, m1, m2)
    usems = (sem_u0, sem_u1, sem_u2)
    msems = (sem_m0, sem_m1, sem_m2)

    def uslice(j):
        return idx_u.at[j // 2, pl.ds((j % 2) * CH, CH)]

    def mslice(j):
        return idx_m.at[j // 2, pl.ds((j % 2) * CH, CH)]

    def fire(j):
        p = j % NBUF
        return (
            pltpu.async_copy(uemb.at[uslice(j)], ubufs[p], usems[p]),
            pltpu.async_copy(memb.at[mslice(j)], mbufs[p], msems[p]),
        )

    pending = {0: fire(0), 1: fire(1)}

    bias_copies = []
    for j in range(IPW):
        bias_copies.append(
            pltpu.async_copy(ubias.at[idx_u.at[j]], bu_all.at[j], sem_b))
    for j in range(IPW):
        bias_copies.append(
            pltpu.async_copy(mbias.at[idx_m.at[j]], bm_all.at[j], sem_b))

    accs = [jnp.zeros((L,), jnp.float32) for _ in range(NSL)]
    for j in range(NCHUNK):
        p = j % NBUF
        cu, cm = pending.pop(j)
        cu.wait()
        cm.wait()
        if j + 2 < NCHUNK:
            pending[j + 2] = fire(j + 2)
        ub, mb = ubufs[p], mbufs[p]

        def body(r, a):
            return tuple(
                a[k] + ub[r, pl.ds(k * L, L)] * mb[r, pl.ds(k * L, L)]
                for k in range(NSL))

        accs = list(lax.fori_loop(0, CH, body, tuple(accs)))

    acc = accs[0]
    for k in range(1, NSL):
        acc = acc + accs[k]
    acc_v[...] = acc
    pltpu.sync_copy(acc_v, parts_out.at[wid])

    for c in bias_copies:
        c.wait()
    pltpu.sync_copy(bu_all, ubg_out.at[pl.ds(rbase, IPW)])
    pltpu.sync_copy(bm_all, mbg_out.at[pl.ds(rbase, IPW)])


_sc_gather_dot = functools.partial(
    pl.kernel,
    out_type=(
        jax.ShapeDtypeStruct((NW, L), jnp.float32),
        jax.ShapeDtypeStruct((IDX_ROWS, 128), jnp.float32),
        jax.ShapeDtypeStruct((IDX_ROWS, 128), jnp.float32),
    ),
    mesh=plsc.VectorSubcoreMesh(core_axis_name="c", subcore_axis_name="s"),
    scratch_types=[
        pltpu.VMEM((IPW, 128), jnp.int32),
        pltpu.VMEM((IPW, 128), jnp.int32),
        pltpu.VMEM((CH, E), jnp.float32),
        pltpu.VMEM((CH, E), jnp.float32),
        pltpu.VMEM((CH, E), jnp.float32),
        pltpu.VMEM((CH, E), jnp.float32),
        pltpu.VMEM((CH, E), jnp.float32),
        pltpu.VMEM((CH, E), jnp.float32),
        pltpu.VMEM((IPW, 128), jnp.float32),
        pltpu.VMEM((IPW, 128), jnp.float32),
        pltpu.VMEM((L,), jnp.float32),
        pltpu.SemaphoreType.DMA,
        pltpu.SemaphoreType.DMA,
        pltpu.SemaphoreType.DMA,
        pltpu.SemaphoreType.DMA,
        pltpu.SemaphoreType.DMA,
        pltpu.SemaphoreType.DMA,
        pltpu.SemaphoreType.DMA,
    ],
)(_sc_body)


def _combine_body(parts_ref, ub_ref, mb_ref, o_ref):
    s = jnp.sum(parts_ref[...])
    o_ref[...] = jax.nn.sigmoid(ub_ref[...] + mb_ref[...] + s)


def kernel(inputs, user_emb, user_bias_tbl, movie_emb, movie_bias_tbl):
    idx = inputs.astype(jnp.int32)
    uidx = idx[:, 0].reshape(IDX_ROWS, 128)
    midx = idx[:, 1].reshape(IDX_ROWS, 128)
    parts, ubg, mbg = _sc_gather_dot(
        uidx, midx, user_emb, movie_emb,
        jnp.squeeze(user_bias_tbl, 1), jnp.squeeze(movie_bias_tbl, 1))
    out = pl.pallas_call(
        _combine_body,
        out_shape=jax.ShapeDtypeStruct((IDX_ROWS, 128), jnp.float32),
    )(parts, ubg, mbg)
    return out.reshape(B, 1)


# speedup vs baseline: 1.0093x; 1.0093x over previous
import functools

import jax
import jax.numpy as jnp
from jax import lax
from jax.experimental import pallas as pl
from jax.experimental.pallas import tpu as pltpu
from jax.experimental.pallas import tpu_sc as plsc

NC = 2
NS = 16
L = 16
NW = NC * NS
B = 16384
E = 128
NUSERS = 100100
BPW = B // NW
CH = 64
NCHUNK = BPW // CH
NBUF = 3
IDX_ROWS = B // 128
NSL = E // L
IPW = BPW // 128


def _sc_body(uidx, midx, uemb, memb, bias_cat,
             parts_out, ubg_out, mbg_out,
             idx_u, idx_m, u0, m0, u1, m1, u2, m2, bu_all, bm_all, acc_v,
             sem_u0, sem_m0, sem_u1, sem_m1, sem_u2, sem_m2, sem_b):
    wid = lax.axis_index("s") * NC + lax.axis_index("c")
    rbase = wid * IPW
    pltpu.sync_copy(uidx.at[pl.ds(rbase, IPW)], idx_u)
    pltpu.sync_copy(midx.at[pl.ds(rbase, IPW)], idx_m)

    ubufs = (u0, u1, u2)
    mbufs = (m0, m1, m2)
    usems = (sem_u0, sem_u1, sem_u2)
    msems = (sem_m0, sem_m1, sem_m2)

    def uslice(j):
        return idx_u.at[j // 2, pl.ds((j % 2) * CH, CH)]

    def mslice(j):
        return idx_m.at[j // 2, pl.ds((j % 2) * CH, CH)]

    def fire(j):
        p = j % NBUF
        return (
            pltpu.async_copy(uemb.at[uslice(j)], ubufs[p], usems[p]),
            pltpu.async_copy(memb.at[mslice(j)], mbufs[p], msems[p]),
        )

    pending = {0: fire(0), 1: fire(1)}

    bias_copies = []
    for j in range(IPW):
        bias_copies.append(
            pltpu.async_copy(bias_cat.at[idx_u.at[j]], bu_all.at[j], sem_b))

    accs = [jnp.zeros((L,), jnp.float32) for _ in range(NSL)]
    for j in range(NCHUNK):
        p = j % NBUF
        cu, cm = pending.pop(j)
        cu.wait()
        cm.wait()
        if j + 2 < NCHUNK:
            pending[j + 2] = fire(j + 2)
        if j == NCHUNK - 1:
            for jj in range(IPW):
                for t in range(128 // L):
                    sl = pl.ds(t * L, L)
                    idx_m[jj, sl] = idx_m[jj, sl] + NUSERS
            for jj in range(IPW):
                bias_copies.append(
                    pltpu.async_copy(
                        bias_cat.at[idx_m.at[jj]], bm_all.at[jj], sem_b))
        ub, mb = ubufs[p], mbufs[p]

        def body(r, a):
            return tuple(
                a[k] + ub[r, pl.ds(k * L, L)] * mb[r, pl.ds(k * L, L)]
                for k in range(NSL))

        accs = list(lax.fori_loop(0, CH, body, tuple(accs)))

    acc = accs[0]
    for k in range(1, NSL):
        acc = acc + accs[k]
    acc_v[...] = acc
    pltpu.sync_copy(acc_v, parts_out.at[wid])

    for c in bias_copies:
        c.wait()
    pltpu.sync_copy(bu_all, ubg_out.at[pl.ds(rbase, IPW)])
    pltpu.sync_copy(bm_all, mbg_out.at[pl.ds(rbase, IPW)])


_sc_gather_dot = functools.partial(
    pl.kernel,
    out_type=(
        jax.ShapeDtypeStruct((NW, L), jnp.float32),
        jax.ShapeDtypeStruct((IDX_ROWS, 128), jnp.float32),
        jax.ShapeDtypeStruct((IDX_ROWS, 128), jnp.float32),
    ),
    mesh=plsc.VectorSubcoreMesh(core_axis_name="c", subcore_axis_name="s"),
    scratch_types=[
        pltpu.VMEM((IPW, 128), jnp.int32),
        pltpu.VMEM((IPW, 128), jnp.int32),
        pltpu.VMEM((CH, E), jnp.float32),
        pltpu.VMEM((CH, E), jnp.float32),
        pltpu.VMEM((CH, E), jnp.float32),
        pltpu.VMEM((CH, E), jnp.float32),
        pltpu.VMEM((CH, E), jnp.float32),
        pltpu.VMEM((CH, E), jnp.float32),
        pltpu.VMEM((IPW, 128), jnp.float32),
        pltpu.VMEM((IPW, 128), jnp.float32),
        pltpu.VMEM((L,), jnp.float32),
        pltpu.SemaphoreType.DMA,
        pltpu.SemaphoreType.DMA,
        pltpu.SemaphoreType.DMA,
        pltpu.SemaphoreType.DMA,
        pltpu.SemaphoreType.DMA,
        pltpu.SemaphoreType.DMA,
        pltpu.SemaphoreType.DMA,
    ],
)(_sc_body)


def _combine_body(parts_ref, ub_ref, mb_ref, o_ref):
    s = jnp.sum(parts_ref[...])
    o_ref[...] = jax.nn.sigmoid(ub_ref[...] + mb_ref[...] + s)


def kernel(inputs, user_emb, user_bias_tbl, movie_emb, movie_bias_tbl):
    idx = inputs.astype(jnp.int32)
    uidx = idx[:, 0].reshape(IDX_ROWS, 128)
    midx = idx[:, 1].reshape(IDX_ROWS, 128)
    bias_cat = jnp.concatenate([user_bias_tbl[:, 0], movie_bias_tbl[:, 0]])
    parts, ubg, mbg = _sc_gather_dot(uidx, midx, user_emb, movie_emb, bias_cat)
    out = pl.pallas_call(
        _combine_body,
        out_shape=jax.ShapeDtypeStruct((IDX_ROWS, 128), jnp.float32),
    )(parts, ubg, mbg)
    return out.reshape(B, 1)


# speedup vs baseline: 1.0113x; 1.0020x over previous
import functools

import jax
import jax.numpy as jnp
from jax import lax
from jax.experimental import pallas as pl
from jax.experimental.pallas import tpu as pltpu
from jax.experimental.pallas import tpu_sc as plsc

NC = 2
NS = 16
L = 16
NW = NC * NS
B = 16384
E = 128
NUSERS = 100100
BPW = B // NW
CH = 64
NCHUNK = BPW // CH
NBUF = 3
IDX_ROWS = B // 128
NSL = E // L
IPW = BPW // 128


def _sc_body(uidx, midx, uemb, memb, bias_cat,
             parts_out, ubg_out, mbg_out,
             idx_u, idx_m, u0, m0, u1, m1, u2, m2, bu_all, bm_all, acc_v,
             sem_u0, sem_m0, sem_u1, sem_m1, sem_u2, sem_m2, sem_b):
    wid = lax.axis_index("s") * NC + lax.axis_index("c")
    rbase = wid * IPW
    pltpu.sync_copy(uidx.at[pl.ds(rbase, IPW)], idx_u)
    pltpu.sync_copy(midx.at[pl.ds(rbase, IPW)], idx_m)

    ubufs = (u0, u1, u2)
    mbufs = (m0, m1, m2)
    usems = (sem_u0, sem_u1, sem_u2)
    msems = (sem_m0, sem_m1, sem_m2)

    def uslice(j):
        return idx_u.at[j // 2, pl.ds((j % 2) * CH, CH)]

    def mslice(j):
        return idx_m.at[j // 2, pl.ds((j % 2) * CH, CH)]

    def fire(j):
        p = j % NBUF
        return (
            pltpu.async_copy(uemb.at[uslice(j)], ubufs[p], usems[p]),
            pltpu.async_copy(memb.at[mslice(j)], mbufs[p], msems[p]),
        )

    pending = {0: fire(0), 1: fire(1)}

    bias_copies = []
    accs = [jnp.zeros((L,), jnp.float32) for _ in range(NSL)]
    for j in range(NCHUNK):
        p = j % NBUF
        if j + 2 < NCHUNK:
            pending[j + 2] = fire(j + 2)
        cu, cm = pending.pop(j)
        cu.wait()
        cm.wait()
        if j == NCHUNK - 2:
            for jj in range(IPW):
                bias_copies.append(
                    pltpu.async_copy(
                        bias_cat.at[idx_u.at[jj]], bu_all.at[jj], sem_b))
        if j == NCHUNK - 1:
            for jj in range(IPW):
                for t in range(128 // L):
                    sl = pl.ds(t * L, L)
                    idx_m[jj, sl] = idx_m[jj, sl] + NUSERS
            for jj in range(IPW):
                bias_copies.append(
                    pltpu.async_copy(
                        bias_cat.at[idx_m.at[jj]], bm_all.at[jj], sem_b))
        ub, mb = ubufs[p], mbufs[p]

        def body(r, a):
            return tuple(
                a[k] + ub[r, pl.ds(k * L, L)] * mb[r, pl.ds(k * L, L)]
                for k in range(NSL))

        accs = list(lax.fori_loop(0, CH, body, tuple(accs)))

    acc = accs[0]
    for k in range(1, NSL):
        acc = acc + accs[k]
    acc_v[...] = acc
    pltpu.sync_copy(acc_v, parts_out.at[wid])

    for c in bias_copies:
        c.wait()
    pltpu.sync_copy(bu_all, ubg_out.at[pl.ds(rbase, IPW)])
    pltpu.sync_copy(bm_all, mbg_out.at[pl.ds(rbase, IPW)])


_sc_gather_dot = functools.partial(
    pl.kernel,
    out_type=(
        jax.ShapeDtypeStruct((NW, L), jnp.float32),
        jax.ShapeDtypeStruct((IDX_ROWS, 128), jnp.float32),
        jax.ShapeDtypeStruct((IDX_ROWS, 128), jnp.float32),
    ),
    mesh=plsc.VectorSubcoreMesh(core_axis_name="c", subcore_axis_name="s"),
    scratch_types=[
        pltpu.VMEM((IPW, 128), jnp.int32),
        pltpu.VMEM((IPW, 128), jnp.int32),
        pltpu.VMEM((CH, E), jnp.float32),
        pltpu.VMEM((CH, E), jnp.float32),
        pltpu.VMEM((CH, E), jnp.float32),
        pltpu.VMEM((CH, E), jnp.float32),
        pltpu.VMEM((CH, E), jnp.float32),
        pltpu.VMEM((CH, E), jnp.float32),
        pltpu.VMEM((IPW, 128), jnp.float32),
        pltpu.VMEM((IPW, 128), jnp.float32),
        pltpu.VMEM((L,), jnp.float32),
        pltpu.SemaphoreType.DMA,
        pltpu.SemaphoreType.DMA,
        pltpu.SemaphoreType.DMA,
        pltpu.SemaphoreType.DMA,
        pltpu.SemaphoreType.DMA,
        pltpu.SemaphoreType.DMA,
        pltpu.SemaphoreType.DMA,
    ],
)(_sc_body)


def _combine_body(parts_ref, ub_ref, mb_ref, o_ref):
    s = jnp.sum(parts_ref[...])
    o_ref[...] = jax.nn.sigmoid(ub_ref[...] + mb_ref[...] + s)


def kernel(inputs, user_emb, user_bias_tbl, movie_emb, movie_bias_tbl):
    idx = inputs.astype(jnp.int32)
    uidx = idx[:, 0].reshape(IDX_ROWS, 128)
    midx = idx[:, 1].reshape(IDX_ROWS, 128)
    bias_cat = jnp.concatenate([user_bias_tbl[:, 0], movie_bias_tbl[:, 0]])
    parts, ubg, mbg = _sc_gather_dot(uidx, midx, user_emb, movie_emb, bias_cat)
    out = pl.pallas_call(
        _combine_body,
        out_shape=jax.ShapeDtypeStruct((IDX_ROWS, 128), jnp.float32),
    )(parts, ubg, mbg)
    return out.reshape(B, 1)
